# baseline (speedup 1.0000x reference)
import jax
import jax.numpy as jnp
from jax.experimental import pallas as pl
from jax.experimental.pallas import tpu as pltpu

MODEL_DIM = 768
HIDDEN = 256
NUM_GROUPS = 2
EPG = 4
NUM_EXPERTS = NUM_GROUPS * EPG
TOP_K = 2
NG = NUM_GROUPS + NUM_EXPERTS

TOKEN_TILE = 1024


def _moe_kernel(x_ref, si_ref, so_ref, w1_ref, w2_ref, gg_ref, eg_ref,
                gb_ref, eb_ref, out_ref,
                si_st, so_st, w1_st, w2_st, gg_st, eg_st, gb_st, eb_st,
                w_in_s, w_out_s, gates_s, bias_s, sems):
    i = pl.program_id(0)

    cp_si = pltpu.make_async_copy(si_ref, si_st, sems.at[0])
    cp_so = pltpu.make_async_copy(so_ref, so_st, sems.at[1])
    cp_w1 = pltpu.make_async_copy(w1_ref, w1_st, sems.at[2])
    cp_w2 = pltpu.make_async_copy(w2_ref, w2_st, sems.at[3])
    cp_gg = pltpu.make_async_copy(gg_ref, gg_st, sems.at[4])
    cp_eg = pltpu.make_async_copy(eg_ref, eg_st, sems.at[5])
    cp_gb = pltpu.make_async_copy(gb_ref, gb_st, sems.at[6])
    cp_eb = pltpu.make_async_copy(eb_ref, eb_st, sems.at[7])

    @pl.when(i == 0)
    def _start():
        cp_gg.start()
        cp_eg.start()
        cp_gb.start()
        cp_eb.start()
        cp_si.start()
        cp_w1.start()
        cp_so.start()
        cp_w2.start()
        cp_gg.wait()
        cp_eg.wait()
        cp_gb.wait()
        cp_eb.wait()
        gates_s[:, :NUM_GROUPS] = gg_st[...]
        gates_s[:, NUM_GROUPS:] = eg_st[...]
        bias_s[:NUM_GROUPS, :] = gb_st[...]
        bias_s[NUM_GROUPS:, :] = eb_st[...]

    xt = x_ref[...]

    logits_t = jax.lax.dot_general(
        gates_s[...], xt, (((0,), (1,)), ((), ())),
        preferred_element_type=jnp.float32)
    logits_t = logits_t + bias_s[...]
    gl = logits_t[:NUM_GROUPS, :]
    el = logits_t[NUM_GROUPS:, :]

    gm = jnp.max(gl, axis=0, keepdims=True)
    ge = jnp.exp(gl - gm)
    gp = ge / jnp.sum(ge, axis=0, keepdims=True)
    p0 = gp[0:1, :]
    p1 = gp[1:2, :]
    g1 = (p1 > p0).astype(jnp.int32)
    g_prob = jnp.where(g1 > 0, p1, p0)

    row = jax.lax.broadcasted_iota(jnp.int32, el.shape, 0)
    in_group = (row // EPG) == g1
    neg = jnp.float32(-1e30)
    el_m = jnp.where(in_group, el, neg)
    em = jnp.max(el_m, axis=0, keepdims=True)
    ee = jnp.exp(el_m - em)
    ep = ee / jnp.sum(ee, axis=0, keepdims=True)
    p_exp = ep * g_prob

    rank = jnp.zeros_like(p_exp)
    for j in range(NUM_EXPERTS):
        pj = p_exp[j:j + 1, :]
        rank = rank + (pj > p_exp).astype(jnp.float32)
        rank = rank + ((pj == p_exp) & (row > j)).astype(jnp.float32)
    c_t = jnp.where(rank < TOP_K, p_exp, 0.0)
    c = c_t.T.astype(jnp.bfloat16)

    @pl.when(i == 0)
    def _prep_in():
        cp_si.wait()
        cp_w1.wait()
        w_in_s[:, :2 * HIDDEN] = si_st[...].astype(jnp.bfloat16)
        w_in_s[:, 2 * HIDDEN:] = w1_st[...].astype(jnp.bfloat16)

    xb = xt.astype(jnp.bfloat16)
    big1 = jnp.dot(xb, w_in_s[...], preferred_element_type=jnp.float32)
    hs = big1[:, :2 * HIDDEN]
    h1 = big1[:, 2 * HIDDEN:]
    a = hs[:, :HIDDEN]
    b = hs[:, HIDDEN:]
    hid = ((a * jax.nn.sigmoid(a)) * b).astype(jnp.bfloat16)
    a1 = h1[:, :HIDDEN]
    b1 = h1[:, HIDDEN:]
    h_all = ((a1 * jax.nn.sigmoid(a1)) * b1).astype(jnp.bfloat16)

    @pl.when(i == 0)
    def _prep_out():
        cp_so.wait()
        cp_w2.wait()
        w_out_s[:HIDDEN, :] = so_st[...].astype(jnp.bfloat16)
        w_out_s[HIDDEN:, :] = w2_st[...].astype(jnp.bfloat16)

    parts = [hid] + [h_all * c[:, e:e + 1] for e in range(NUM_EXPERTS)]
    scaled = jnp.concatenate(parts, axis=1)
    out_ref[...] = jnp.dot(scaled, w_out_s[...],
                           preferred_element_type=jnp.float32)


def kernel(x, shared_in_w, shared_out_w, w1_shared_w, w2_expert_w,
           group_gate_w, expert_gate_w, group_bias, expert_bias):
    Bb, Tt, C = x.shape
    S = Bb * Tt
    flat = x.reshape(S, C)
    w2_flat = w2_expert_w.reshape(NUM_EXPERTS * HIDDEN, C)
    gb = group_bias.reshape(NUM_GROUPS, 1)
    eb = expert_bias.reshape(NUM_EXPERTS, 1)

    grid = (S // TOKEN_TILE,)
    hbm = pl.BlockSpec(memory_space=pltpu.MemorySpace.HBM)

    out = pl.pallas_call(
        _moe_kernel,
        grid=grid,
        in_specs=[
            pl.BlockSpec((TOKEN_TILE, C), lambda i: (i, 0)),
            hbm, hbm, hbm, hbm, hbm, hbm, hbm, hbm,
        ],
        out_specs=pl.BlockSpec((TOKEN_TILE, C), lambda i: (i, 0)),
        out_shape=jax.ShapeDtypeStruct((S, C), jnp.float32),
        scratch_shapes=[
            pltpu.VMEM((C, 2 * HIDDEN), jnp.float32),
            pltpu.VMEM((HIDDEN, C), jnp.float32),
            pltpu.VMEM((C, 2 * HIDDEN), jnp.float32),
            pltpu.VMEM((NUM_EXPERTS * HIDDEN, C), jnp.float32),
            pltpu.VMEM((C, NUM_GROUPS), jnp.float32),
            pltpu.VMEM((C, NUM_EXPERTS), jnp.float32),
            pltpu.VMEM((NUM_GROUPS, 1), jnp.float32),
            pltpu.VMEM((NUM_EXPERTS, 1), jnp.float32),
            pltpu.VMEM((C, 4 * HIDDEN), jnp.bfloat16),
            pltpu.VMEM(((1 + NUM_EXPERTS) * HIDDEN, C), jnp.bfloat16),
            pltpu.VMEM((C, NG), jnp.float32),
            pltpu.VMEM((NG, 1), jnp.float32),
            pltpu.SemaphoreType.DMA((8,)),
        ],
    )(flat, shared_in_w, shared_out_w, w1_shared_w, w2_flat,
      group_gate_w, expert_gate_w, gb, eb)

    return out.reshape(Bb, Tt, C)


# speedup vs baseline: 1.0462x; 1.0462x over previous
import jax
import jax.numpy as jnp
from jax.experimental import pallas as pl
from jax.experimental.pallas import tpu as pltpu

MODEL_DIM = 768
HIDDEN = 256
NUM_GROUPS = 2
EPG = 4
NUM_EXPERTS = NUM_GROUPS * EPG
TOP_K = 2
NG = NUM_GROUPS + NUM_EXPERTS

TOKEN_TILE = 1024


def _moe_kernel(x_ref, si_ref, so_ref, w1_ref, w2_ref, gg_ref, eg_ref,
                gb_ref, eb_ref, out_ref, w_in_s, w_out_s, gates_s, bias_s):
    i = pl.program_id(0)

    @pl.when(i == 0)
    def _prep():
        w_in_s[:, :2 * HIDDEN] = si_ref[...].astype(jnp.bfloat16)
        w_in_s[:, 2 * HIDDEN:] = w1_ref[...].astype(jnp.bfloat16)
        w_out_s[:HIDDEN, :] = so_ref[...].astype(jnp.bfloat16)
        w_out_s[HIDDEN:, :] = w2_ref[...].astype(jnp.bfloat16)
        gates_s[:, :NUM_GROUPS] = gg_ref[...]
        gates_s[:, NUM_GROUPS:] = eg_ref[...]
        bias_s[:NUM_GROUPS, :] = gb_ref[...]
        bias_s[NUM_GROUPS:, :] = eb_ref[...]

    xt = x_ref[...]

    logits_t = jax.lax.dot_general(
        gates_s[...], xt, (((0,), (1,)), ((), ())),
        preferred_element_type=jnp.float32)
    logits_t = logits_t + bias_s[...]
    gl = logits_t[:NUM_GROUPS, :]
    el = logits_t[NUM_GROUPS:, :]

    gm = jnp.max(gl, axis=0, keepdims=True)
    ge = jnp.exp(gl - gm)
    gp = ge / jnp.sum(ge, axis=0, keepdims=True)
    p0 = gp[0:1, :]
    p1 = gp[1:2, :]
    g1 = (p1 > p0).astype(jnp.int32)
    g_prob = jnp.where(g1 > 0, p1, p0)

    row = jax.lax.broadcasted_iota(jnp.int32, el.shape, 0)
    in_group = (row // EPG) == g1
    neg = jnp.float32(-1e30)
    el_m = jnp.where(in_group, el, neg)
    em = jnp.max(el_m, axis=0, keepdims=True)
    ee = jnp.exp(el_m - em)
    ep = ee / jnp.sum(ee, axis=0, keepdims=True)
    p_exp = ep * g_prob

    rank = jnp.zeros_like(p_exp)
    for j in range(NUM_EXPERTS):
        pj = p_exp[j:j + 1, :]
        rank = rank + (pj > p_exp).astype(jnp.float32)
        rank = rank + ((pj == p_exp) & (row > j)).astype(jnp.float32)
    c_t = jnp.where(rank < TOP_K, p_exp, 0.0)
    c = c_t.T.astype(jnp.bfloat16)

    xb = xt.astype(jnp.bfloat16)
    big1 = jnp.dot(xb, w_in_s[...], preferred_element_type=jnp.float32)
    hs = big1[:, :2 * HIDDEN]
    h1 = big1[:, 2 * HIDDEN:]
    a = hs[:, :HIDDEN]
    b = hs[:, HIDDEN:]
    hid = ((a * jax.nn.sigmoid(a)) * b).astype(jnp.bfloat16)
    a1 = h1[:, :HIDDEN]
    b1 = h1[:, HIDDEN:]
    h_all = ((a1 * jax.nn.sigmoid(a1)) * b1).astype(jnp.bfloat16)

    parts = [hid] + [h_all * c[:, e:e + 1] for e in range(NUM_EXPERTS)]
    scaled = jnp.concatenate(parts, axis=1)
    out_ref[...] = jnp.dot(scaled, w_out_s[...],
                           preferred_element_type=jnp.float32)


def kernel(x, shared_in_w, shared_out_w, w1_shared_w, w2_expert_w,
           group_gate_w, expert_gate_w, group_bias, expert_bias):
    Bb, Tt, C = x.shape
    S = Bb * Tt
    flat = x.reshape(S, C)
    w2_flat = w2_expert_w.reshape(NUM_EXPERTS * HIDDEN, C)
    gb = group_bias.reshape(NUM_GROUPS, 1)
    eb = expert_bias.reshape(NUM_EXPERTS, 1)

    grid = (S // TOKEN_TILE,)
    full = lambda *shape: pl.BlockSpec(shape, lambda i: (0,) * len(shape))

    out = pl.pallas_call(
        _moe_kernel,
        grid=grid,
        in_specs=[
            pl.BlockSpec((TOKEN_TILE, C), lambda i: (i, 0)),
            full(C, 2 * HIDDEN),
            full(HIDDEN, C),
            full(C, 2 * HIDDEN),
            full(NUM_EXPERTS * HIDDEN, C),
            full(C, NUM_GROUPS),
            full(C, NUM_EXPERTS),
            full(NUM_GROUPS, 1),
            full(NUM_EXPERTS, 1),
        ],
        out_specs=pl.BlockSpec((TOKEN_TILE, C), lambda i: (i, 0)),
        out_shape=jax.ShapeDtypeStruct((S, C), jnp.float32),
        scratch_shapes=[
            pltpu.VMEM((C, 4 * HIDDEN), jnp.bfloat16),
            pltpu.VMEM(((1 + NUM_EXPERTS) * HIDDEN, C), jnp.bfloat16),
            pltpu.VMEM((C, NG), jnp.float32),
            pltpu.VMEM((NG, 1), jnp.float32),
        ],
    )(flat, shared_in_w, shared_out_w, w1_shared_w, w2_flat,
      group_gate_w, expert_gate_w, gb, eb)

    return out.reshape(Bb, Tt, C)
